# baseline (speedup 1.0000x reference)
import functools

import jax
import jax.numpy as jnp
from jax.experimental import pallas as pl
from jax.experimental.pallas import tpu as pltpu


def _ce_kernel(x_ref, lab_ref, m_ref, slen_ref, end_ref, out_ref,
               acc_sum, acc_cnt, *, n_rows, mp1, v):
    i = pl.program_id(0)
    nb = pl.num_programs(0)

    @pl.when(i == 0)
    def _init():
        acc_sum[0, 0] = 0.0
        acc_cnt[0, 0] = 0.0

    x = x_ref[0]
    lab = lab_ref[0]
    m = m_ref[0]
    slen = slen_ref[0, 0, 0]

    r = jax.lax.broadcasted_iota(jnp.int32, (n_rows, 1), 0)
    jj = r // mp1
    kk = r - mp1 * jj
    valid = (jj < slen) & (kk <= m)
    lab_full = jnp.where(kk == m, end_ref[0, 0, 0], lab)

    mx = jnp.max(x, axis=1, keepdims=True)
    s = jnp.sum(jnp.exp(x - mx), axis=1, keepdims=True)
    lse = mx + jnp.log(s)
    lane = jax.lax.broadcasted_iota(jnp.int32, (n_rows, v), 1)
    t = jnp.sum(jnp.where(lane == lab_full, x, 0.0), axis=1, keepdims=True)
    nll = lse - t

    acc_sum[0, 0] += jnp.sum(jnp.where(valid, nll, 0.0))
    acc_cnt[0, 0] += jnp.sum(jnp.where(valid, 1.0, 0.0))

    @pl.when(i == nb - 1)
    def _fin():
        out_ref[0, 0] = acc_sum[0, 0] / acc_cnt[0, 0]


def kernel(labels, logits, seq_length, m_length_matrix, med_num, END_TOKEN):
    B, S, M = labels.shape
    Mp1 = logits.shape[2]
    V = logits.shape[3]
    n_rows = S * Mp1

    logits_r = logits.reshape(B, n_rows, V)
    pad = jnp.zeros((B, S, Mp1 - M), dtype=labels.dtype)
    lab_flat = jnp.concatenate([labels, pad], axis=2).reshape(B, n_rows, 1)
    m_flat = jnp.broadcast_to(
        m_length_matrix[:, :, None], (B, S, Mp1)).reshape(B, n_rows, 1)
    slen = seq_length.astype(jnp.int32).reshape(B, 1, 1)
    end_tok = jnp.asarray(END_TOKEN, dtype=jnp.int32).reshape(1, 1, 1)

    body = functools.partial(_ce_kernel, n_rows=n_rows, mp1=Mp1, v=V)

    out = pl.pallas_call(
        body,
        grid=(B,),
        in_specs=[
            pl.BlockSpec((1, n_rows, V), lambda i: (i, 0, 0)),
            pl.BlockSpec((1, n_rows, 1), lambda i: (i, 0, 0)),
            pl.BlockSpec((1, n_rows, 1), lambda i: (i, 0, 0)),
            pl.BlockSpec((1, 1, 1), lambda i: (i, 0, 0)),
            pl.BlockSpec((1, 1, 1), lambda i: (0, 0, 0)),
        ],
        out_specs=pl.BlockSpec(memory_space=pltpu.MemorySpace.SMEM),
        out_shape=jax.ShapeDtypeStruct((1, 1), jnp.float32),
        scratch_shapes=[
            pltpu.SMEM((1, 1), jnp.float32),
            pltpu.SMEM((1, 1), jnp.float32),
        ],
    )(logits_r, lab_flat, m_flat, slen, end_tok)
    return out[0, 0]
